# baseline (speedup 1.0000x reference)
import functools

import jax
import jax.numpy as jnp
from jax.experimental import pallas as pl
from jax.experimental.pallas import tpu as pltpu

_F32 = jnp.float32


def _leaky(x):
    return jnp.where(x >= 0, x, 0.02 * x)


def _norm_leaky(v):
    c = v.shape[0]
    m = jnp.mean(v, axis=0, keepdims=True)
    xc = v - m
    denom = jnp.sqrt(jnp.sum(xc * xc, axis=0, keepdims=True)) * (c ** -0.5) + 1e-5
    return _leaky(xc / denom)


def _moe_body(x_ref, g_ref, w_ref, h_ref, loss_ref, onehot_s, *, nexp, out_scale,
              round_out):
    dt = pl.program_id(0)
    e = pl.program_id(1)

    @pl.when((dt == 0) & (e == 0))
    def _router():
        x = x_ref[...]
        logits = jax.lax.dot_general(
            g_ref[...], x, (((1,), (0,)), ((), ())),
            preferred_element_type=_F32)
        best = logits[0:1, :]
        arg = jnp.zeros_like(best, dtype=jnp.int32)
        for k in range(1, nexp):
            row = logits[k:k + 1, :]
            gt = row > best
            arg = jnp.where(gt, k, arg)
            best = jnp.where(gt, row, best)
        eidx = jax.lax.broadcasted_iota(jnp.int32, logits.shape, 0)
        onehot = (eidx == arg).astype(_F32)
        onehot_s[...] = onehot
        ex = jnp.exp(logits - best)
        gates = ex / jnp.sum(ex, axis=0, keepdims=True)
        loss = jnp.sum(jnp.mean(gates, axis=1, keepdims=True)
                       * jnp.mean(onehot, axis=1, keepdims=True),
                       axis=0, keepdims=True)
        loss_ref[...] = loss

    mask = onehot_s[pl.ds(e, 1), :]
    masked = x_ref[...] * mask * out_scale
    contrib = jax.lax.dot_general(
        w_ref[0], masked, (((0,), (0,)), ((), ())),
        preferred_element_type=_F32)

    @pl.when(e == 0)
    def _init():
        h_ref[...] = contrib

    @pl.when(e > 0)
    def _acc():
        h_ref[...] += contrib

    if round_out:
        @pl.when(e == nexp - 1)
        def _round():
            h_ref[...] = h_ref[...].astype(jnp.bfloat16).astype(_F32)


def _moe(x, gate, w, out_scale, round_out=False):
    c, s = x.shape
    nexp, _, d = w.shape
    dtile = d // 2 if d % 2304 == 0 else d
    ndt = d // dtile
    h, loss = pl.pallas_call(
        functools.partial(_moe_body, nexp=nexp, out_scale=out_scale,
                          round_out=round_out),
        grid=(ndt, nexp),
        in_specs=[
            pl.BlockSpec((c, s), lambda dt, e: (0, 0)),
            pl.BlockSpec((nexp, c), lambda dt, e: (0, 0)),
            pl.BlockSpec((1, c, dtile), lambda dt, e: (e, 0, dt)),
        ],
        out_specs=[
            pl.BlockSpec((dtile, s), lambda dt, e: (dt, 0)),
            pl.BlockSpec((1, 1), lambda dt, e: (0, 0)),
        ],
        out_shape=[
            jax.ShapeDtypeStruct((d, s), _F32),
            jax.ShapeDtypeStruct((1, 1), _F32),
        ],
        scratch_shapes=[pltpu.VMEM((nexp, s), _F32)],
    )(x, gate, w)
    return h, loss


def _stage2_body(h_ref, o_ref):
    c, s = o_ref.shape
    nch = s // 128
    x3 = h_ref[0:c, :].reshape(c, nch, 128)
    cum3 = x3
    for _ in range(127):
        cum3 = x3 + jnp.concatenate(
            [jnp.zeros((c, nch, 1), _F32), cum3[:, :, :128 - 1]], axis=2)
    totals = cum3[:, :, 127]
    sh1 = jnp.concatenate(
        [jnp.zeros((c, 1), _F32), totals[:, :nch - 1]], axis=1)
    offs = sh1
    for _ in range(nch - 2):
        offs = sh1 + jnp.concatenate(
            [jnp.zeros((c, 1), _F32), offs[:, :nch - 1]], axis=1)
    cum = (cum3 + offs[:, :, None]).reshape(c, s)
    div = jax.lax.broadcasted_iota(jnp.int32, (1, s), 1).astype(_F32) + 1.0
    v = cum / div * h_ref[c:2 * c, :] + h_ref[2 * c:3 * c, :]
    o_ref[...] = _norm_leaky(v)


def _stage2(h):
    d, s = h.shape
    c = d // 3
    return pl.pallas_call(
        _stage2_body,
        out_shape=jax.ShapeDtypeStruct((c, s), _F32),
    )(h)


def _conv_body(x_ref, w_ref, y_ref, *, ksize):
    cg, s = x_ref.shape
    taps = []
    for k in range(ksize):
        sh = ksize - 1 - k
        if sh > 0:
            taps.append(jnp.concatenate(
                [jnp.zeros((cg, sh), _F32), x_ref[:, :s - sh]], axis=1))
        else:
            taps.append(x_ref[...])
    xcol = jnp.stack(taps, axis=1).reshape(cg * ksize, s)
    y_ref[...] = jax.lax.dot_general(
        w_ref[...], xcol, (((1,), (0,)), ((), ())),
        preferred_element_type=_F32)


def _conv(x, w1r, ksize, groups):
    c, s = x.shape
    o = w1r.shape[0]
    cg = c // groups
    og = o // groups
    return pl.pallas_call(
        functools.partial(_conv_body, ksize=ksize),
        grid=(groups,),
        in_specs=[
            pl.BlockSpec((cg, s), lambda g: (g, 0)),
            pl.BlockSpec((og, cg * ksize), lambda g: (g, 0)),
        ],
        out_specs=pl.BlockSpec((og, s), lambda g: (g, 0)),
        out_shape=jax.ShapeDtypeStruct((o, s), _F32),
    )(x, w1r)


def _stage4_body(y_ref, o_ref):
    c = o_ref.shape[0]
    s0 = y_ref[0:c, :]
    s1 = y_ref[c:2 * c, :]
    sh = y_ref[2 * c:3 * c, :]
    o_ref[...] = _norm_leaky(s0 * s1 + sh)


def _stage4(y):
    d, s = y.shape
    c = d // 3
    return pl.pallas_call(
        _stage4_body,
        out_shape=jax.ShapeDtypeStruct((c, s), _F32),
    )(y)


def kernel(inp, w0_gate, w0, w1, w2_gate, w2):
    x0 = inp[0]
    g0 = w0_gate[:, :, 0]
    g2 = w2_gate[:, :, 0]
    ksize = w1.shape[-1]
    w1r = w1.reshape(w1.shape[0], -1)

    h, loss0 = _moe(x0, g0, w0, 1.0, round_out=True)
    x1 = _stage2(h)
    y = _conv(x1, w1r, ksize, 4)
    x2 = _stage4(y)
    out, loss1 = _moe(x2, g2, w2, 0.125)
    return loss0.reshape(()), loss1.reshape(()), out[None]


# speedup vs baseline: 1.1730x; 1.1730x over previous
import functools

import jax
import jax.numpy as jnp
from jax.experimental import pallas as pl
from jax.experimental.pallas import tpu as pltpu

_F32 = jnp.float32


def _leaky(x):
    return jnp.where(x >= 0, x, 0.02 * x)


def _norm_leaky(v):
    c = v.shape[0]
    m = jnp.mean(v, axis=0, keepdims=True)
    xc = v - m
    denom = jnp.sqrt(jnp.sum(xc * xc, axis=0, keepdims=True)) * (c ** -0.5) + 1e-5
    return _leaky(xc / denom)


def _moe_body(x_ref, g_ref, w_ref, h_ref, loss_ref, onehot_s, x_s, *, nexp,
              out_scale, round_out, gate_stage):
    dt = pl.program_id(0)
    e = pl.program_id(1)

    @pl.when((dt == 0) & (e == 0))
    def _router():
        if gate_stage:
            c = x_s.shape[0]
            s0 = x_ref[0:c, :]
            s1 = x_ref[c:2 * c, :]
            sh = x_ref[2 * c:3 * c, :]
            x_s[...] = _norm_leaky(s0 * s1 + sh)
            x = x_s[...]
        else:
            x = x_ref[...]
        logits = jax.lax.dot_general(
            g_ref[...], x, (((1,), (0,)), ((), ())),
            preferred_element_type=_F32)
        best = logits[0:1, :]
        arg = jnp.zeros_like(best, dtype=jnp.int32)
        for k in range(1, nexp):
            row = logits[k:k + 1, :]
            gt = row > best
            arg = jnp.where(gt, k, arg)
            best = jnp.where(gt, row, best)
        eidx = jax.lax.broadcasted_iota(jnp.int32, logits.shape, 0)
        onehot = (eidx == arg).astype(_F32)
        onehot_s[...] = onehot
        ex = jnp.exp(logits - best)
        gates = ex / jnp.sum(ex, axis=0, keepdims=True)
        loss = jnp.sum(jnp.mean(gates, axis=1, keepdims=True)
                       * jnp.mean(onehot, axis=1, keepdims=True),
                       axis=0, keepdims=True)
        loss_ref[...] = loss

    mask = onehot_s[pl.ds(e, 1), :]
    xin = x_s[...] if gate_stage else x_ref[...]
    masked = xin * mask * out_scale
    contrib = jax.lax.dot_general(
        w_ref[0], masked, (((0,), (0,)), ((), ())),
        preferred_element_type=_F32)

    @pl.when(e == 0)
    def _init():
        h_ref[...] = contrib

    @pl.when(e > 0)
    def _acc():
        h_ref[...] += contrib

    if round_out:
        @pl.when(e == nexp - 1)
        def _round():
            h_ref[...] = h_ref[...].astype(jnp.bfloat16).astype(_F32)


def _moe(x, gate, w, out_scale, round_out=False, gate_stage=False):
    cin, s = x.shape
    nexp, c, d = w.shape[0], w.shape[1], w.shape[2]
    dtile = d // 2 if d % 2304 == 0 else d
    ndt = d // dtile
    h, loss = pl.pallas_call(
        functools.partial(_moe_body, nexp=nexp, out_scale=out_scale,
                          round_out=round_out, gate_stage=gate_stage),
        grid=(ndt, nexp),
        in_specs=[
            pl.BlockSpec((cin, s), lambda dt, e: (0, 0)),
            pl.BlockSpec((nexp, c), lambda dt, e: (0, 0)),
            pl.BlockSpec((1, c, dtile), lambda dt, e: (e, 0, dt)),
        ],
        out_specs=[
            pl.BlockSpec((dtile, s), lambda dt, e: (dt, 0)),
            pl.BlockSpec((1, 1), lambda dt, e: (0, 0)),
        ],
        out_shape=[
            jax.ShapeDtypeStruct((d, s), _F32),
            jax.ShapeDtypeStruct((1, 1), _F32),
        ],
        scratch_shapes=[pltpu.VMEM((nexp, s), _F32),
                        pltpu.VMEM((c, s), _F32)],
    )(x, gate, w)
    return h, loss


def _stage2_body(h_ref, o_ref):
    c, s = o_ref.shape
    nch = s // 128
    x3 = h_ref[0:c, :].reshape(c, nch, 128)
    cum3 = x3
    for _ in range(127):
        cum3 = x3 + jnp.concatenate(
            [jnp.zeros((c, nch, 1), _F32), cum3[:, :, :128 - 1]], axis=2)
    totals = cum3[:, :, 127]
    sh1 = jnp.concatenate(
        [jnp.zeros((c, 1), _F32), totals[:, :nch - 1]], axis=1)
    offs = sh1
    for _ in range(nch - 2):
        offs = sh1 + jnp.concatenate(
            [jnp.zeros((c, 1), _F32), offs[:, :nch - 1]], axis=1)
    cum = (cum3 + offs[:, :, None]).reshape(c, s)
    div = jax.lax.broadcasted_iota(jnp.int32, (1, s), 1).astype(_F32) + 1.0
    v = cum / div * h_ref[c:2 * c, :] + h_ref[2 * c:3 * c, :]
    o_ref[...] = _norm_leaky(v)


def _stage2(h):
    d, s = h.shape
    c = d // 3
    return pl.pallas_call(
        _stage2_body,
        out_shape=jax.ShapeDtypeStruct((c, s), _F32),
    )(h)


def _conv_body(x_ref, w_ref, y_ref, *, ksize):
    cg, s = x_ref.shape
    acc = None
    for k in range(ksize):
        sh = ksize - 1 - k
        if sh > 0:
            xs = jnp.concatenate(
                [jnp.zeros((cg, sh), _F32), x_ref[:, :s - sh]], axis=1)
        else:
            xs = x_ref[...]
        t = jax.lax.dot_general(
            w_ref[k], xs, (((1,), (0,)), ((), ())),
            preferred_element_type=_F32)
        acc = t if acc is None else acc + t
    y_ref[...] = acc


def _conv(x, w1t, groups):
    c, s = x.shape
    ksize, o, cg = w1t.shape
    og = o // groups
    return pl.pallas_call(
        functools.partial(_conv_body, ksize=ksize),
        grid=(groups,),
        in_specs=[
            pl.BlockSpec((cg, s), lambda g: (g, 0)),
            pl.BlockSpec((ksize, og, cg), lambda g: (0, g, 0)),
        ],
        out_specs=pl.BlockSpec((og, s), lambda g: (g, 0)),
        out_shape=jax.ShapeDtypeStruct((o, s), _F32),
    )(x, w1t)


def _stage4_body(y_ref, o_ref):
    c = o_ref.shape[0]
    s0 = y_ref[0:c, :]
    s1 = y_ref[c:2 * c, :]
    sh = y_ref[2 * c:3 * c, :]
    o_ref[...] = _norm_leaky(s0 * s1 + sh)


def _stage4(y):
    d, s = y.shape
    c = d // 3
    return pl.pallas_call(
        _stage4_body,
        out_shape=jax.ShapeDtypeStruct((c, s), _F32),
    )(y)


def kernel(inp, w0_gate, w0, w1, w2_gate, w2):
    x0 = inp[0]
    g0 = w0_gate[:, :, 0]
    g2 = w2_gate[:, :, 0]
    w1t = jnp.transpose(w1, (2, 0, 1))

    h, loss0 = _moe(x0, g0, w0, 1.0, round_out=True)
    x1 = _stage2(h)
    y = _conv(x1, w1t, 4)
    out, loss1 = _moe(y, g2, w2, 0.125, gate_stage=True)
    return loss0.reshape(()), loss1.reshape(()), out[None]
